# baseline (speedup 1.0000x reference)
import functools

import jax
import jax.numpy as jnp
from jax import lax
from jax.experimental import pallas as pl
from jax.experimental.pallas import tpu as pltpu
from jax.experimental.pallas import tpu_sc as plsc

_BATCH = 16384
_RULES = 64
_ROW = 16
_LANES = 16

_info = plsc.get_sparse_core_info()
_NC = _info.num_cores
_NW = _NC * _info.num_subcores
_BPW = _BATCH // _NW


_CHUNKS = 8
_RPC = _BPW // _CHUNKS


def _sc_body(x_hbm, out_hbm, x_v, out_v, sem):
    wid = lax.axis_index("s") * _NC + lax.axis_index("c")
    base = wid * _BPW
    pltpu.sync_copy(x_hbm.at[pl.ds(base * _ROW, _BPW * _ROW)], x_v)

    lane = lax.broadcasted_iota(jnp.int32, (_LANES,), 0)
    half = lane >> 3
    c_off = 8 + (lane & 7)

    copies = []
    for ch in range(_CHUNKS):
        @plsc.parallel_loop(ch * _RPC, (ch + 1) * _RPC, unroll=8)
        def row(r):
            b = r * _ROW
            c = plsc.load_gather(x_v, [b + c_off])
            for k in range(4):
                a = plsc.load_gather(x_v, [b + 2 * k + half])
                out_v[pl.ds(r * _RULES + k * _LANES, _LANES)] = jnp.minimum(a, c)

        off = ch * _RPC * _RULES
        copies.append(pltpu.async_copy(
            out_v.at[pl.ds(off, _RPC * _RULES)],
            out_hbm.at[pl.ds(base * _RULES + off, _RPC * _RULES)],
            sem))
    for cp in copies:
        cp.wait()


@jax.jit
def kernel(x):
    xf = x.reshape(_BATCH * _ROW)
    out = pl.kernel(
        _sc_body,
        out_type=jax.ShapeDtypeStruct((_BATCH * _RULES,), jnp.float32),
        mesh=plsc.VectorSubcoreMesh(core_axis_name="c", subcore_axis_name="s"),
        compiler_params=pltpu.CompilerParams(needs_layout_passes=False),
        scratch_types=[
            pltpu.VMEM((_BPW * _ROW,), jnp.float32),
            pltpu.VMEM((_BPW * _RULES,), jnp.float32),
            pltpu.SemaphoreType.DMA,
        ],
    )(xf)
    return out.reshape(_BATCH, _RULES)


# speedup vs baseline: 1.0209x; 1.0209x over previous
import functools

import jax
import jax.numpy as jnp
from jax import lax
from jax.experimental import pallas as pl
from jax.experimental.pallas import tpu as pltpu
from jax.experimental.pallas import tpu_sc as plsc

_BATCH = 16384
_RULES = 64
_ROW = 16
_LANES = 16

_info = plsc.get_sparse_core_info()
_NC = _info.num_cores
_NW = _NC * _info.num_subcores
_BPW = _BATCH // _NW


_CHUNKS = 8
_RPC = _BPW // _CHUNKS


def _sc_body(x_hbm, out_hbm, x_v, out_v, sem):
    wid = lax.axis_index("s") * _NC + lax.axis_index("c")
    base = wid * _BPW
    pltpu.sync_copy(x_hbm.at[pl.ds(base * _ROW, _BPW * _ROW)], x_v)

    lane = lax.broadcasted_iota(jnp.int32, (_LANES,), 0)
    half = lane >> 3
    c_off = 8 + (lane & 7)
    a_pats = [2 * k + half for k in range(4)]

    copies = []
    for ch in range(_CHUNKS):
        @plsc.parallel_loop(ch * _RPC, (ch + 1) * _RPC, unroll=8)
        def row(r):
            xrow = x_v.at[pl.ds(r * _ROW, _ROW)]
            c = plsc.load_gather(xrow, [c_off])
            for k in range(4):
                a = plsc.load_gather(xrow, [a_pats[k]])
                out_v[pl.ds(r * _RULES + k * _LANES, _LANES)] = jnp.minimum(a, c)

        off = ch * _RPC * _RULES
        copies.append(pltpu.async_copy(
            out_v.at[pl.ds(off, _RPC * _RULES)],
            out_hbm.at[pl.ds(base * _RULES + off, _RPC * _RULES)],
            sem))
    for cp in copies:
        cp.wait()


@jax.jit
def kernel(x):
    xf = x.reshape(_BATCH * _ROW)
    out = pl.kernel(
        _sc_body,
        out_type=jax.ShapeDtypeStruct((_BATCH * _RULES,), jnp.float32),
        mesh=plsc.VectorSubcoreMesh(core_axis_name="c", subcore_axis_name="s"),
        compiler_params=pltpu.CompilerParams(needs_layout_passes=False),
        scratch_types=[
            pltpu.VMEM((_BPW * _ROW,), jnp.float32),
            pltpu.VMEM((_BPW * _RULES,), jnp.float32),
            pltpu.SemaphoreType.DMA,
        ],
    )(xf)
    return out.reshape(_BATCH, _RULES)


# speedup vs baseline: 1.0232x; 1.0022x over previous
import jax
import jax.numpy as jnp
from jax import lax
from jax.experimental import pallas as pl
from jax.experimental.pallas import tpu as pltpu
from jax.experimental.pallas import tpu_sc as plsc

_BATCH = 16384
_RULES = 64
_ROW = 16
_LANES = 16

_info = plsc.get_sparse_core_info()
_NC = _info.num_cores
_NW = _NC * _info.num_subcores
_BPW = _BATCH // _NW


_CHUNKS = 8
_RPC = _BPW // _CHUNKS


def _sc_body(x_hbm, out_hbm, x_v, out_v, sem):
    wid = lax.axis_index("s") * _NC + lax.axis_index("c")
    base = wid * _BPW
    pltpu.sync_copy(x_hbm.at[pl.ds(base * _ROW, _BPW * _ROW)], x_v)

    lane = lax.broadcasted_iota(jnp.int32, (_LANES,), 0)
    half = lane >> 3
    c_off = 8 + (lane & 7)
    a_pats = [2 * k + half for k in range(4)]

    copies = []
    for ch in range(_CHUNKS):
        @plsc.parallel_loop(ch * _RPC, (ch + 1) * _RPC, unroll=8)
        def row(r):
            xrow = x_v.at[pl.ds(r * _ROW, _ROW)]
            c = plsc.load_gather(xrow, [c_off])
            for k in range(4):
                a = plsc.load_gather(xrow, [a_pats[k]])
                out_v[pl.ds(r * _RULES + k * _LANES, _LANES)] = jnp.minimum(a, c)

        off = ch * _RPC * _RULES
        copies.append(pltpu.async_copy(
            out_v.at[pl.ds(off, _RPC * _RULES)],
            out_hbm.at[pl.ds(base * _RULES + off, _RPC * _RULES)],
            sem))
    for cp in copies:
        cp.wait()


@jax.jit
def kernel(x):
    xf = x.reshape(_BATCH * _ROW)
    out = pl.kernel(
        _sc_body,
        out_type=jax.ShapeDtypeStruct((_BATCH * _RULES,), jnp.float32),
        mesh=plsc.VectorSubcoreMesh(core_axis_name="c", subcore_axis_name="s"),
        compiler_params=pltpu.CompilerParams(needs_layout_passes=False),
        scratch_types=[
            pltpu.VMEM((_BPW * _ROW,), jnp.float32),
            pltpu.VMEM((_BPW * _RULES,), jnp.float32),
            pltpu.SemaphoreType.DMA,
        ],
    )(xf)
    return out.reshape(_BATCH, _RULES)
